# baseline (speedup 1.0000x reference)
import functools

import jax
import jax.numpy as jnp
from jax import lax
from jax.experimental import pallas as pl
from jax.experimental.pallas import tpu as pltpu
from jax.experimental.pallas import tpu_sc as plsc

_G = 16
_NTILES = 32
_CH = 64
_NB = 4
_ZR = 16
_R0 = 160
_HV = 2
_PAD_ROWS = 64


def _premlp_body(x_ref, w1_ref, b1_ref, w2_ref, b2_ref, o_ref):
    h = jnp.dot(x_ref[...], w1_ref[...], preferred_element_type=jnp.float32)
    h = jnp.maximum(h + b1_ref[...], 0.0)
    o = jnp.dot(h, w2_ref[...], preferred_element_type=jnp.float32)
    o_ref[...] = jnp.maximum(o + b2_ref[...], 0.0)


def _premlp(x, w1, b1, w2, b2, blk):
    n, d = x.shape
    h = w1.shape[1]
    return pl.pallas_call(
        _premlp_body,
        grid=(n // blk,),
        in_specs=[
            pl.BlockSpec((blk, d), lambda i: (i, 0)),
            pl.BlockSpec((d, h), lambda i: (0, 0)),
            pl.BlockSpec((1, h), lambda i: (0, 0)),
            pl.BlockSpec((h, h), lambda i: (0, 0)),
            pl.BlockSpec((1, h), lambda i: (0, 0)),
        ],
        out_specs=pl.BlockSpec((blk, h), lambda i: (i, 0)),
        out_shape=jax.ShapeDtypeStruct((n, h), jnp.float32),
    )(x, w1, b1, w2, b2)


def _layer_body(h_ref, agg_ref, batch_ref, eps_ref, w1_ref, b1_ref, w2_ref,
                b2_ref, hout_ref, pool_ref):
    i = pl.program_id(0)
    eps = eps_ref[0, 0]
    a = (1.0 + eps) * h_ref[...] + agg_ref[...]
    z = jnp.dot(a, w1_ref[...], preferred_element_type=jnp.float32)
    z = jnp.maximum(z + b1_ref[...], 0.0)
    hn = jnp.dot(z, w2_ref[...], preferred_element_type=jnp.float32) + b2_ref[...]
    hout_ref[...] = hn
    blk = h_ref.shape[0]
    seg = batch_ref[0]
    ids = lax.broadcasted_iota(jnp.int32, (_G, blk), 0)
    m = (seg == ids).astype(jnp.float32)
    p = jnp.dot(m, hn, preferred_element_type=jnp.float32)

    @pl.when(i == 0)
    def _():
        pool_ref[...] = p

    @pl.when(i != 0)
    def _():
        pool_ref[...] += p


def _layer_tc(h, agg, batch3, eps, w1, b1, w2, b2, blk):
    n, d = h.shape
    return pl.pallas_call(
        _layer_body,
        grid=(n // blk,),
        in_specs=[
            pl.BlockSpec((blk, d), lambda i: (i, 0)),
            pl.BlockSpec((blk, d), lambda i: (i, 0)),
            pl.BlockSpec((1, 1, blk), lambda i: (i, 0, 0)),
            pl.BlockSpec((1, 1), lambda i: (0, 0)),
            pl.BlockSpec((d, d), lambda i: (0, 0)),
            pl.BlockSpec((1, d), lambda i: (0, 0)),
            pl.BlockSpec((d, d), lambda i: (0, 0)),
            pl.BlockSpec((1, d), lambda i: (0, 0)),
        ],
        out_specs=[
            pl.BlockSpec((blk, d), lambda i: (i, 0)),
            pl.BlockSpec((_G, d), lambda i: (0, 0)),
        ],
        out_shape=[
            jax.ShapeDtypeStruct((n, d), jnp.float32),
            jax.ShapeDtypeStruct((_G, d), jnp.float32),
        ],
    )(h, agg, batch3, eps, w1, b1, w2, b2)


def _make_sc_scatter(n, d):
    h0 = _R0 // _HV
    ng0 = h0 * 2 // _NB
    cpr = 128 // _CH
    assert h0 * _HV == _R0 and ng0 * _NB == h0 * 2
    assert h0 % 8 == 0 and _NB % cpr == 0
    assert d % 16 == 0 and _CH % 16 == 0
    n_acc = n + _PAD_ROWS
    rpt = (n // 16) // 8 * 8
    rem_out = n - 16 * rpt
    rem_z = n_acc - 16 * rpt
    nz = rpt // _ZR
    assert rpt % _ZR == 0 and rem_z % _ZR == 0
    nz_rem = rem_z // _ZR
    mesh = plsc.VectorSubcoreMesh(core_axis_name="c", subcore_axis_name="s")

    @functools.partial(
        pl.kernel,
        mesh=mesh,
        out_type=jax.ShapeDtypeStruct((n, d), jnp.float32),
        scratch_types=[
            pltpu.VMEM((_R0 // _HV, 128), jnp.int32),
        ] + [pltpu.VMEM((_CH, d), jnp.float32) for _ in range(_NB)]
        + [pltpu.VMEM((_CH,), jnp.int32) for _ in range(_NB)]
        + [pltpu.VMEM((_CH,), jnp.int32),
           pltpu.VMEM((_ZR, d), jnp.float32),
           pltpu.VMEM_SHARED((n_acc, d), jnp.float32)]
        + [pltpu.SemaphoreType.DMA for _ in range(_NB)]
        + [pltpu.SemaphoreType.DMA],
    )
    def sc_scatter(h_hbm, idx_hbm, out_hbm, idx_v, *refs):
        rows = refs[:_NB]
        sstg = refs[_NB:2 * _NB]
        dstg, zbuf, agg_sh = refs[2 * _NB:2 * _NB + 3]
        sems = refs[2 * _NB + 3:3 * _NB + 3]
        zsem = refs[3 * _NB + 3]
        c = lax.axis_index("c")
        s = lax.axis_index("s")
        base = s * rpt

        @pl.when(c == 0)
        def _zero_fill():
            def _zrow(r, carry):
                for j in range(d // 16):
                    zbuf[r, pl.ds(j * 16, 16)] = jnp.zeros((16,), jnp.float32)
                return carry
            lax.fori_loop(0, _ZR, _zrow, 0)
            for j in range(nz):
                pltpu.async_copy(zbuf, agg_sh.at[pl.ds(base + j * _ZR, _ZR)],
                                 zsem)

            @pl.when(s == 15)
            def _():
                for j in range(nz_rem):
                    pltpu.async_copy(
                        zbuf, agg_sh.at[pl.ds(16 * rpt + j * _ZR, _ZR)], zsem)

        def _unpack_src(r, l0, b):
            for j in range(_CH // 16):
                p = idx_v[r, pl.ds(l0 + j * 16, 16)]
                sstg[b][pl.ds(j * 16, 16)] = p & 0xFFFF

        def _unpack_dst(r, l0):
            for j in range(_CH // 16):
                p = idx_v[r, pl.ds(l0 + j * 16, 16)]
                dstg[pl.ds(j * 16, 16)] = p >> 16

        @pl.when(c == 0)
        def _main():
            for j in range(nz):
                pltpu.make_async_copy(
                    zbuf, agg_sh.at[pl.ds(base + j * _ZR, _ZR)], zsem).wait()

            @pl.when(s == 15)
            def _():
                for j in range(nz_rem):
                    pltpu.make_async_copy(
                        zbuf, agg_sh.at[pl.ds(16 * rpt + j * _ZR, _ZR)],
                        zsem).wait()
            plsc.subcore_barrier()

            for t in range(_HV):
                half_row = s * _R0 + t * h0
                pltpu.sync_copy(idx_hbm.at[pl.ds(half_row, h0)], idx_v)

                for b in range(_NB):
                    _unpack_src(b // cpr, (b % cpr) * _CH, b)
                    pltpu.async_copy(h_hbm.at[sstg[b]], rows[b], sems[b])

                def _group(g, carry):
                    for b in range(_NB):
                        r = (_NB // cpr) * g + b // cpr
                        l0 = (b % cpr) * _CH
                        pltpu.make_async_copy(
                            h_hbm.at[sstg[b]], rows[b], sems[b]).wait()
                        _unpack_dst(r, l0)
                        pltpu.sync_copy(rows[b], agg_sh.at[dstg], add=True)

                        @pl.when(g < ng0 - 1)
                        def _(r=r, l0=l0, b=b):
                            _unpack_src(r + _NB // cpr, l0, b)
                            pltpu.async_copy(
                                h_hbm.at[sstg[b]], rows[b], sems[b])
                    return carry
                lax.fori_loop(0, ng0, _group, 0)

            plsc.subcore_barrier()

            @pl.when(s < 15)
            def _():
                pltpu.sync_copy(agg_sh.at[pl.ds(base, rpt)],
                                out_hbm.at[pl.ds(base, rpt)])

            @pl.when(s == 15)
            def _():
                pltpu.sync_copy(agg_sh.at[pl.ds(base, rpt + rem_out)],
                                out_hbm.at[pl.ds(base, rpt + rem_out)])

    return sc_scatter


def kernel(x, params, edge_index, batch):
    n, d = x.shape
    e = edge_index.shape[1]
    blk = 2000

    e_pad = 16 * _R0 * 128
    pad = e_pad - e
    src = edge_index[0].astype(jnp.int32)
    dst = edge_index[1].astype(jnp.int32)
    src = jnp.concatenate([src, jnp.zeros((pad,), jnp.int32)])
    dst = jnp.concatenate(
        [dst, n + (jnp.arange(pad, dtype=jnp.int32) % _PAD_ROWS)])
    packed = (src | (dst << 16)).reshape(16 * _R0, 128)
    batch3 = batch.astype(jnp.int32).reshape(n // blk, 1, blk)

    h = _premlp(x,
                params["pre1_w"], params["pre1_b"].reshape(1, -1),
                params["pre2_w"], params["pre2_b"].reshape(1, -1), blk)

    sc_scatter = _make_sc_scatter(n, d)
    pooled = []
    for lp in params["layers"]:
        agg = sc_scatter(h, packed)
        h, p = _layer_tc(h, agg, batch3, lp["eps"].reshape(1, 1),
                         lp["w1"], lp["b1"].reshape(1, -1),
                         lp["w2"], lp["b2"].reshape(1, -1), blk)
        pooled.append(p)
    return jnp.concatenate(pooled, axis=1)


# speedup vs baseline: 2.7631x; 2.7631x over previous
import functools

import jax
import jax.numpy as jnp
from jax import lax
from jax.experimental import pallas as pl
from jax.experimental.pallas import tpu as pltpu
from jax.experimental.pallas import tpu_sc as plsc

_G = 16
_NTILES = 32
_CH = 64
_NB = 4
_ZR = 16
_R0 = 160
_HV = 2
_PAD_ROWS = 64


def _premlp_body(x_ref, w1_ref, b1_ref, w2_ref, b2_ref, o_ref):
    h = jnp.dot(x_ref[...], w1_ref[...], preferred_element_type=jnp.float32)
    h = jnp.maximum(h + b1_ref[...], 0.0)
    o = jnp.dot(h, w2_ref[...], preferred_element_type=jnp.float32)
    o_ref[...] = jnp.maximum(o + b2_ref[...], 0.0)


def _premlp(x, w1, b1, w2, b2, blk):
    n, d = x.shape
    h = w1.shape[1]
    return pl.pallas_call(
        _premlp_body,
        grid=(n // blk,),
        in_specs=[
            pl.BlockSpec((blk, d), lambda i: (i, 0)),
            pl.BlockSpec((d, h), lambda i: (0, 0)),
            pl.BlockSpec((1, h), lambda i: (0, 0)),
            pl.BlockSpec((h, h), lambda i: (0, 0)),
            pl.BlockSpec((1, h), lambda i: (0, 0)),
        ],
        out_specs=pl.BlockSpec((blk, h), lambda i: (i, 0)),
        out_shape=jax.ShapeDtypeStruct((n, h), jnp.float32),
    )(x, w1, b1, w2, b2)


def _layer_body(h_ref, agg_ref, batch_ref, eps_ref, w1_ref, b1_ref, w2_ref,
                b2_ref, hout_ref, pool_ref):
    i = pl.program_id(0)
    eps = eps_ref[0, 0]
    a = (1.0 + eps) * h_ref[...] + agg_ref[...]
    z = jnp.dot(a, w1_ref[...], preferred_element_type=jnp.float32)
    z = jnp.maximum(z + b1_ref[...], 0.0)
    hn = jnp.dot(z, w2_ref[...], preferred_element_type=jnp.float32) + b2_ref[...]
    hout_ref[...] = hn
    blk = h_ref.shape[0]
    seg = batch_ref[0]
    ids = lax.broadcasted_iota(jnp.int32, (_G, blk), 0)
    m = (seg == ids).astype(jnp.float32)
    p = jnp.dot(m, hn, preferred_element_type=jnp.float32)

    @pl.when(i == 0)
    def _():
        pool_ref[...] = p

    @pl.when(i != 0)
    def _():
        pool_ref[...] += p


def _layer_tc(h, agg, batch3, eps, w1, b1, w2, b2, blk):
    n, d = h.shape
    return pl.pallas_call(
        _layer_body,
        grid=(n // blk,),
        in_specs=[
            pl.BlockSpec((blk, d), lambda i: (i, 0)),
            pl.BlockSpec((blk, d), lambda i: (i, 0)),
            pl.BlockSpec((1, 1, blk), lambda i: (i, 0, 0)),
            pl.BlockSpec((1, 1), lambda i: (0, 0)),
            pl.BlockSpec((d, d), lambda i: (0, 0)),
            pl.BlockSpec((1, d), lambda i: (0, 0)),
            pl.BlockSpec((d, d), lambda i: (0, 0)),
            pl.BlockSpec((1, d), lambda i: (0, 0)),
        ],
        out_specs=[
            pl.BlockSpec((blk, d), lambda i: (i, 0)),
            pl.BlockSpec((_G, d), lambda i: (0, 0)),
        ],
        out_shape=[
            jax.ShapeDtypeStruct((n, d), jnp.float32),
            jax.ShapeDtypeStruct((_G, d), jnp.float32),
        ],
    )(h, agg, batch3, eps, w1, b1, w2, b2)


def _make_sc_scatter(n, d):
    h0 = _R0 // _HV
    ng0 = h0 * 2 // _NB
    cpr = 128 // _CH
    assert h0 * _HV == _R0 and ng0 * _NB == h0 * 2
    assert h0 % 8 == 0 and _NB % cpr == 0
    assert d % 16 == 0 and _CH % 16 == 0
    n_acc = n + _PAD_ROWS
    rpt = (n // 16) // 8 * 8
    rem_out = n - 16 * rpt
    rem_z = n_acc - 16 * rpt
    nz = rpt // _ZR
    assert rpt % _ZR == 0 and rem_z % _ZR == 0
    nz_rem = rem_z // _ZR
    mesh = plsc.VectorSubcoreMesh(core_axis_name="c", subcore_axis_name="s")

    @functools.partial(
        pl.kernel,
        mesh=mesh,
        out_type=jax.ShapeDtypeStruct((n, d), jnp.float32),
        scratch_types=[
            pltpu.VMEM((_R0 // _HV, 128), jnp.int32),
        ] + [pltpu.VMEM((_CH, d), jnp.float32) for _ in range(_NB)]
        + [pltpu.VMEM((_CH,), jnp.int32) for _ in range(_NB)]
        + [pltpu.VMEM((_CH,), jnp.int32),
           pltpu.VMEM((_ZR, d), jnp.float32),
           pltpu.VMEM_SHARED((n_acc, d), jnp.float32)]
        + [pltpu.SemaphoreType.DMA for _ in range(_NB)]
        + [pltpu.SemaphoreType.DMA],
    )
    def sc_scatter(h_hbm, idx_hbm, out_hbm, idx_v, *refs):
        rows = refs[:_NB]
        sstg = refs[_NB:2 * _NB]
        dstg, zbuf, agg_sh = refs[2 * _NB:2 * _NB + 3]
        sems = refs[2 * _NB + 3:3 * _NB + 3]
        zsem = refs[3 * _NB + 3]
        c = lax.axis_index("c")
        s = lax.axis_index("s")
        base = s * rpt

        @pl.when(c == 0)
        def _zero_fill():
            def _zrow(r, carry):
                for j in range(d // 16):
                    zbuf[r, pl.ds(j * 16, 16)] = jnp.zeros((16,), jnp.float32)
                return carry
            lax.fori_loop(0, _ZR, _zrow, 0)
            for j in range(nz):
                pltpu.async_copy(zbuf, agg_sh.at[pl.ds(base + j * _ZR, _ZR)],
                                 zsem)

            @pl.when(s == 15)
            def _():
                for j in range(nz_rem):
                    pltpu.async_copy(
                        zbuf, agg_sh.at[pl.ds(16 * rpt + j * _ZR, _ZR)], zsem)

        def _unpack_src(r, l0, b):
            for j in range(_CH // 16):
                p = idx_v[r, pl.ds(l0 + j * 16, 16)]
                sstg[b][pl.ds(j * 16, 16)] = p & 0xFFFF

        def _unpack_dst(r, l0):
            for j in range(_CH // 16):
                p = idx_v[r, pl.ds(l0 + j * 16, 16)]
                dstg[pl.ds(j * 16, 16)] = p >> 16

        @pl.when(c == 0)
        def _main():
            for j in range(nz):
                pltpu.make_async_copy(
                    zbuf, agg_sh.at[pl.ds(base + j * _ZR, _ZR)], zsem).wait()

            @pl.when(s == 15)
            def _():
                for j in range(nz_rem):
                    pltpu.make_async_copy(
                        zbuf, agg_sh.at[pl.ds(16 * rpt + j * _ZR, _ZR)],
                        zsem).wait()
            plsc.subcore_barrier()

            for t in range(_HV):
                half_row = s * _R0 + t * h0
                pltpu.sync_copy(idx_hbm.at[pl.ds(half_row, h0)], idx_v)

                for b in range(_NB):
                    _unpack_src(b // cpr, (b % cpr) * _CH, b)
                    pltpu.async_copy(h_hbm.at[sstg[b]], rows[b], sems[b])

                def _group(g, carry):
                    for b in range(_NB):
                        r = (_NB // cpr) * g + b // cpr
                        l0 = (b % cpr) * _CH
                        pltpu.make_async_copy(
                            h_hbm.at[sstg[b]], rows[b], sems[b]).wait()
                        _unpack_dst(r, l0)
                        pltpu.sync_copy(rows[b], agg_sh.at[dstg], add=True)

                        @pl.when(g < ng0 - 1)
                        def _(r=r, l0=l0, b=b):
                            _unpack_src(r + _NB // cpr, l0, b)
                            pltpu.async_copy(
                                h_hbm.at[sstg[b]], rows[b], sems[b])
                    return carry
                lax.fori_loop(0, ng0, _group, 0)

            plsc.subcore_barrier()

            @pl.when(s < 15)
            def _():
                pltpu.sync_copy(agg_sh.at[pl.ds(base, rpt)],
                                out_hbm.at[pl.ds(base, rpt)])

            @pl.when(s == 15)
            def _():
                pltpu.sync_copy(agg_sh.at[pl.ds(base, rpt + rem_out)],
                                out_hbm.at[pl.ds(base, rpt + rem_out)])

    return sc_scatter


def kernel(x, params, edge_index, batch):
    n, d = x.shape
    e = edge_index.shape[1]
    blk = 2000

    e_pad = 16 * _R0 * 128
    pad = e_pad - e
    src = edge_index[0].astype(jnp.int32)
    dst = edge_index[1].astype(jnp.int32)
    pad_iota = jnp.arange(pad, dtype=jnp.int32)
    src = jnp.concatenate([src, pad_iota % n])
    dst = jnp.concatenate([dst, n + (pad_iota % _PAD_ROWS)])
    packed = (src | (dst << 16)).reshape(16 * _R0, 128)
    batch3 = batch.astype(jnp.int32).reshape(n // blk, 1, blk)

    h = _premlp(x,
                params["pre1_w"], params["pre1_b"].reshape(1, -1),
                params["pre2_w"], params["pre2_b"].reshape(1, -1), blk)

    sc_scatter = _make_sc_scatter(n, d)
    pooled = []
    for lp in params["layers"]:
        agg = sc_scatter(h, packed)
        h, p = _layer_tc(h, agg, batch3, lp["eps"].reshape(1, 1),
                         lp["w1"], lp["b1"].reshape(1, -1),
                         lp["w2"], lp["b2"].reshape(1, -1), blk)
        pooled.append(p)
    return jnp.concatenate(pooled, axis=1)


# speedup vs baseline: 4.0040x; 1.4491x over previous
import functools

import jax
import jax.numpy as jnp
from jax import lax
from jax.experimental import pallas as pl
from jax.experimental.pallas import tpu as pltpu
from jax.experimental.pallas import tpu_sc as plsc

_G = 16
_NTILES = 32
_CH = 64
_NB = 4
_ZR = 16
_R0 = 96
_R1 = 64
_HV = 2
_PAD_ROWS = 64


def _premlp_body(x_ref, w1_ref, b1_ref, w2_ref, b2_ref, o_ref):
    h = jnp.dot(x_ref[...], w1_ref[...], preferred_element_type=jnp.float32)
    h = jnp.maximum(h + b1_ref[...], 0.0)
    o = jnp.dot(h, w2_ref[...], preferred_element_type=jnp.float32)
    o_ref[...] = jnp.maximum(o + b2_ref[...], 0.0)


def _premlp(x, w1, b1, w2, b2, blk):
    n, d = x.shape
    h = w1.shape[1]
    return pl.pallas_call(
        _premlp_body,
        grid=(n // blk,),
        in_specs=[
            pl.BlockSpec((blk, d), lambda i: (i, 0)),
            pl.BlockSpec((d, h), lambda i: (0, 0)),
            pl.BlockSpec((1, h), lambda i: (0, 0)),
            pl.BlockSpec((h, h), lambda i: (0, 0)),
            pl.BlockSpec((1, h), lambda i: (0, 0)),
        ],
        out_specs=pl.BlockSpec((blk, h), lambda i: (i, 0)),
        out_shape=jax.ShapeDtypeStruct((n, h), jnp.float32),
    )(x, w1, b1, w2, b2)


def _layer_body(h_ref, agg_ref, batch_ref, eps_ref, w1_ref, b1_ref, w2_ref,
                b2_ref, hout_ref, pool_ref):
    i = pl.program_id(0)
    eps = eps_ref[0, 0]
    a = (1.0 + eps) * h_ref[...] + agg_ref[0] + agg_ref[1]
    z = jnp.dot(a, w1_ref[...], preferred_element_type=jnp.float32)
    z = jnp.maximum(z + b1_ref[...], 0.0)
    hn = jnp.dot(z, w2_ref[...], preferred_element_type=jnp.float32) + b2_ref[...]
    hout_ref[...] = hn
    blk = h_ref.shape[0]
    seg = batch_ref[0]
    ids = lax.broadcasted_iota(jnp.int32, (_G, blk), 0)
    m = (seg == ids).astype(jnp.float32)
    p = jnp.dot(m, hn, preferred_element_type=jnp.float32)

    @pl.when(i == 0)
    def _():
        pool_ref[...] = p

    @pl.when(i != 0)
    def _():
        pool_ref[...] += p


def _layer_tc(h, agg, batch3, eps, w1, b1, w2, b2, blk):
    n, d = h.shape
    return pl.pallas_call(
        _layer_body,
        grid=(n // blk,),
        in_specs=[
            pl.BlockSpec((blk, d), lambda i: (i, 0)),
            pl.BlockSpec((2, blk, d), lambda i: (0, i, 0)),
            pl.BlockSpec((1, 1, blk), lambda i: (i, 0, 0)),
            pl.BlockSpec((1, 1), lambda i: (0, 0)),
            pl.BlockSpec((d, d), lambda i: (0, 0)),
            pl.BlockSpec((1, d), lambda i: (0, 0)),
            pl.BlockSpec((d, d), lambda i: (0, 0)),
            pl.BlockSpec((1, d), lambda i: (0, 0)),
        ],
        out_specs=[
            pl.BlockSpec((blk, d), lambda i: (i, 0)),
            pl.BlockSpec((_G, d), lambda i: (0, 0)),
        ],
        out_shape=[
            jax.ShapeDtypeStruct((n, d), jnp.float32),
            jax.ShapeDtypeStruct((_G, d), jnp.float32),
        ],
    )(h, agg, batch3, eps, w1, b1, w2, b2)


def _make_sc_scatter(n, d):
    h0 = _R0 // _HV
    h1 = _R1 // _HV
    ng0 = h0 * 2 // _NB
    ng1 = h1 * 2 // _NB
    cpr = 128 // _CH
    assert h0 * _HV == _R0 and ng0 * _NB == h0 * 2
    assert h1 * _HV == _R1 and ng1 * _NB == h1 * 2
    assert h0 % 8 == 0 and h1 % 8 == 0 and _NB % cpr == 0
    assert d % 16 == 0 and _CH % 16 == 0
    n_acc = n + _PAD_ROWS
    rpt = (n // 16) // 8 * 8
    rem_out = n - 16 * rpt
    rem_z = n_acc - 16 * rpt
    nz = rpt // _ZR
    assert rpt % _ZR == 0 and rem_z % _ZR == 0
    nz_rem = rem_z // _ZR
    mesh = plsc.VectorSubcoreMesh(core_axis_name="c", subcore_axis_name="s")

    @functools.partial(
        pl.kernel,
        mesh=mesh,
        out_type=jax.ShapeDtypeStruct((2, n, d), jnp.float32),
        scratch_types=[
            pltpu.VMEM((_R0 // _HV, 128), jnp.int32),
        ] + [pltpu.VMEM((_CH, d), jnp.float32) for _ in range(_NB)]
        + [pltpu.VMEM((_CH,), jnp.int32) for _ in range(_NB)]
        + [pltpu.VMEM((_CH,), jnp.int32),
           pltpu.VMEM((_ZR, d), jnp.float32),
           pltpu.VMEM_SHARED((n_acc, d), jnp.float32)]
        + [pltpu.SemaphoreType.DMA for _ in range(_NB)]
        + [pltpu.SemaphoreType.DMA],
    )
    def sc_scatter(h_hbm, idx_hbm, out_hbm, idx_v, *refs):
        rows = refs[:_NB]
        sstg = refs[_NB:2 * _NB]
        dstg, zbuf, agg_sh = refs[2 * _NB:2 * _NB + 3]
        sems = refs[2 * _NB + 3:3 * _NB + 3]
        zsem = refs[3 * _NB + 3]
        c = lax.axis_index("c")
        s = lax.axis_index("s")
        base = s * rpt

        def _zrow(r, carry):
            for j in range(d // 16):
                zbuf[r, pl.ds(j * 16, 16)] = jnp.zeros((16,), jnp.float32)
            return carry
        lax.fori_loop(0, _ZR, _zrow, 0)
        for j in range(nz):
            pltpu.async_copy(zbuf, agg_sh.at[pl.ds(base + j * _ZR, _ZR)],
                             zsem)

        @pl.when(s == 15)
        def _():
            for j in range(nz_rem):
                pltpu.async_copy(
                    zbuf, agg_sh.at[pl.ds(16 * rpt + j * _ZR, _ZR)], zsem)

        def _unpack_src(r, l0, b):
            for j in range(_CH // 16):
                p = idx_v[r, pl.ds(l0 + j * 16, 16)]
                sstg[b][pl.ds(j * 16, 16)] = p & 0xFFFF

        def _unpack_dst(r, l0):
            for j in range(_CH // 16):
                p = idx_v[r, pl.ds(l0 + j * 16, 16)]
                dstg[pl.ds(j * 16, 16)] = p >> 16

        for j in range(nz):
            pltpu.make_async_copy(
                zbuf, agg_sh.at[pl.ds(base + j * _ZR, _ZR)], zsem).wait()

        @pl.when(s == 15)
        def _():
            for j in range(nz_rem):
                pltpu.make_async_copy(
                    zbuf, agg_sh.at[pl.ds(16 * rpt + j * _ZR, _ZR)],
                    zsem).wait()
        plsc.subcore_barrier()

        base_row = jnp.where(c == 0, s * _R0, 16 * _R0 + s * _R1)
        hrows = jnp.where(c == 0, h0, h1)
        ngh = jnp.where(c == 0, ng0, ng1)

        for t in range(_HV):
            half_row = base_row + t * hrows

            @pl.when(c == 0)
            def _(half_row=half_row):
                pltpu.sync_copy(idx_hbm.at[pl.ds(half_row, h0)],
                                idx_v.at[pl.ds(0, h0)])

            @pl.when(c == 1)
            def _(half_row=half_row):
                pltpu.sync_copy(idx_hbm.at[pl.ds(half_row, h1)],
                                idx_v.at[pl.ds(0, h1)])

            for b in range(_NB):
                _unpack_src(b // cpr, (b % cpr) * _CH, b)
                pltpu.async_copy(h_hbm.at[sstg[b]], rows[b], sems[b])

            def _group(g, carry):
                for b in range(_NB):
                    r = (_NB // cpr) * g + b // cpr
                    l0 = (b % cpr) * _CH
                    pltpu.make_async_copy(
                        h_hbm.at[sstg[b]], rows[b], sems[b]).wait()
                    _unpack_dst(r, l0)
                    pltpu.sync_copy(rows[b], agg_sh.at[dstg], add=True)

                    @pl.when(g < ngh - 1)
                    def _(r=r, l0=l0, b=b):
                        _unpack_src(r + _NB // cpr, l0, b)
                        pltpu.async_copy(
                            h_hbm.at[sstg[b]], rows[b], sems[b])
                return carry
            lax.fori_loop(0, ngh, _group, 0)

        plsc.subcore_barrier()

        @pl.when(s < 15)
        def _():
            pltpu.sync_copy(agg_sh.at[pl.ds(base, rpt)],
                            out_hbm.at[c].at[pl.ds(base, rpt)])

        @pl.when(s == 15)
        def _():
            pltpu.sync_copy(agg_sh.at[pl.ds(base, rpt + rem_out)],
                            out_hbm.at[c].at[pl.ds(base, rpt + rem_out)])

    return sc_scatter


def kernel(x, params, edge_index, batch):
    n, d = x.shape
    e = edge_index.shape[1]
    blk = 2000

    e_pad = 16 * (_R0 + _R1) * 128
    pad = e_pad - e
    src = edge_index[0].astype(jnp.int32)
    dst = edge_index[1].astype(jnp.int32)
    pad_iota = jnp.arange(pad, dtype=jnp.int32)
    src = jnp.concatenate([src, pad_iota % n])
    dst = jnp.concatenate([dst, n + (pad_iota % _PAD_ROWS)])
    packed = (src | (dst << 16)).reshape(16 * (_R0 + _R1), 128)
    batch3 = batch.astype(jnp.int32).reshape(n // blk, 1, blk)

    h = _premlp(x,
                params["pre1_w"], params["pre1_b"].reshape(1, -1),
                params["pre2_w"], params["pre2_b"].reshape(1, -1), blk)

    sc_scatter = _make_sc_scatter(n, d)
    pooled = []
    for lp in params["layers"]:
        agg = sc_scatter(h, packed)
        h, p = _layer_tc(h, agg, batch3, lp["eps"].reshape(1, 1),
                         lp["w1"], lp["b1"].reshape(1, -1),
                         lp["w2"], lp["b2"].reshape(1, -1), blk)
        pooled.append(p)
    return jnp.concatenate(pooled, axis=1)


# speedup vs baseline: 4.5212x; 1.1292x over previous
import functools

import jax
import jax.numpy as jnp
from jax import lax
from jax.experimental import pallas as pl
from jax.experimental.pallas import tpu as pltpu
from jax.experimental.pallas import tpu_sc as plsc

_G = 16
_NTILES = 32
_CH = 64
_NB = 4
_ZR = 16
_R0 = 80
_R1 = 80
_HV = 2
_PAD_ROWS = 64


def _premlp_body(x_ref, w1_ref, b1_ref, w2_ref, b2_ref, o_ref):
    h = jnp.dot(x_ref[...], w1_ref[...], preferred_element_type=jnp.float32)
    h = jnp.maximum(h + b1_ref[...], 0.0)
    o = jnp.dot(h, w2_ref[...], preferred_element_type=jnp.float32)
    o_ref[...] = jnp.maximum(o + b2_ref[...], 0.0)


def _premlp(x, w1, b1, w2, b2, blk):
    n, d = x.shape
    h = w1.shape[1]
    return pl.pallas_call(
        _premlp_body,
        grid=(n // blk,),
        in_specs=[
            pl.BlockSpec((blk, d), lambda i: (i, 0)),
            pl.BlockSpec((d, h), lambda i: (0, 0)),
            pl.BlockSpec((1, h), lambda i: (0, 0)),
            pl.BlockSpec((h, h), lambda i: (0, 0)),
            pl.BlockSpec((1, h), lambda i: (0, 0)),
        ],
        out_specs=pl.BlockSpec((blk, h), lambda i: (i, 0)),
        out_shape=jax.ShapeDtypeStruct((n, h), jnp.float32),
    )(x, w1, b1, w2, b2)


def _layer_body(h_ref, agg_ref, batch_ref, eps_ref, w1_ref, b1_ref, w2_ref,
                b2_ref, hout_ref, pool_ref):
    i = pl.program_id(0)
    eps = eps_ref[0, 0]
    a = (1.0 + eps) * h_ref[...] + agg_ref[0] + agg_ref[1]
    z = jnp.dot(a, w1_ref[...], preferred_element_type=jnp.float32)
    z = jnp.maximum(z + b1_ref[...], 0.0)
    hn = jnp.dot(z, w2_ref[...], preferred_element_type=jnp.float32) + b2_ref[...]
    hout_ref[...] = hn
    blk = h_ref.shape[0]
    seg = batch_ref[0]
    ids = lax.broadcasted_iota(jnp.int32, (_G, blk), 0)
    m = (seg == ids).astype(jnp.float32)
    p = jnp.dot(m, hn, preferred_element_type=jnp.float32)

    @pl.when(i == 0)
    def _():
        pool_ref[...] = p

    @pl.when(i != 0)
    def _():
        pool_ref[...] += p


def _layer_tc(h, agg, batch3, eps, w1, b1, w2, b2, blk):
    n, d = h.shape
    return pl.pallas_call(
        _layer_body,
        grid=(n // blk,),
        in_specs=[
            pl.BlockSpec((blk, d), lambda i: (i, 0)),
            pl.BlockSpec((2, blk, d), lambda i: (0, i, 0)),
            pl.BlockSpec((1, 1, blk), lambda i: (i, 0, 0)),
            pl.BlockSpec((1, 1), lambda i: (0, 0)),
            pl.BlockSpec((d, d), lambda i: (0, 0)),
            pl.BlockSpec((1, d), lambda i: (0, 0)),
            pl.BlockSpec((d, d), lambda i: (0, 0)),
            pl.BlockSpec((1, d), lambda i: (0, 0)),
        ],
        out_specs=[
            pl.BlockSpec((blk, d), lambda i: (i, 0)),
            pl.BlockSpec((_G, d), lambda i: (0, 0)),
        ],
        out_shape=[
            jax.ShapeDtypeStruct((n, d), jnp.float32),
            jax.ShapeDtypeStruct((_G, d), jnp.float32),
        ],
    )(h, agg, batch3, eps, w1, b1, w2, b2)


def _make_sc_scatter(n, d):
    h0 = _R0 // _HV
    h1 = _R1 // _HV
    ng0 = h0 * 2 // _NB
    ng1 = h1 * 2 // _NB
    cpr = 128 // _CH
    assert h0 * _HV == _R0 and ng0 * _NB == h0 * 2
    assert h1 * _HV == _R1 and ng1 * _NB == h1 * 2
    assert h0 % 8 == 0 and h1 % 8 == 0 and _NB % cpr == 0
    assert d % 16 == 0 and _CH % 16 == 0
    n_acc = n + _PAD_ROWS
    rpt = (n // 16) // 8 * 8
    rem_out = n - 16 * rpt
    rem_z = n_acc - 16 * rpt
    nz = rpt // _ZR
    assert rpt % _ZR == 0 and rem_z % _ZR == 0
    nz_rem = rem_z // _ZR
    mesh = plsc.VectorSubcoreMesh(core_axis_name="c", subcore_axis_name="s")

    @functools.partial(
        pl.kernel,
        mesh=mesh,
        out_type=jax.ShapeDtypeStruct((2, n, d), jnp.float32),
        scratch_types=[
            pltpu.VMEM((_R0 // _HV, 128), jnp.int32),
        ] + [pltpu.VMEM((_CH, d), jnp.float32) for _ in range(_NB)]
        + [pltpu.VMEM((_CH,), jnp.int32) for _ in range(_NB)]
        + [pltpu.VMEM((_CH,), jnp.int32),
           pltpu.VMEM((_ZR, d), jnp.float32),
           pltpu.VMEM_SHARED((n_acc, d), jnp.float32)]
        + [pltpu.SemaphoreType.DMA for _ in range(_NB)]
        + [pltpu.SemaphoreType.DMA],
    )
    def sc_scatter(h_hbm, idx_hbm, out_hbm, idx_v, *refs):
        rows = refs[:_NB]
        sstg = refs[_NB:2 * _NB]
        dstg, zbuf, agg_sh = refs[2 * _NB:2 * _NB + 3]
        sems = refs[2 * _NB + 3:3 * _NB + 3]
        zsem = refs[3 * _NB + 3]
        c = lax.axis_index("c")
        s = lax.axis_index("s")
        base = s * rpt

        def _zrow(r, carry):
            for j in range(d // 16):
                zbuf[r, pl.ds(j * 16, 16)] = jnp.zeros((16,), jnp.float32)
            return carry
        lax.fori_loop(0, _ZR, _zrow, 0)
        for j in range(nz):
            pltpu.async_copy(zbuf, agg_sh.at[pl.ds(base + j * _ZR, _ZR)],
                             zsem)

        @pl.when(s == 15)
        def _():
            for j in range(nz_rem):
                pltpu.async_copy(
                    zbuf, agg_sh.at[pl.ds(16 * rpt + j * _ZR, _ZR)], zsem)

        def _unpack_src(r, l0, b):
            for j in range(_CH // 16):
                p = idx_v[r, pl.ds(l0 + j * 16, 16)]
                sstg[b][pl.ds(j * 16, 16)] = p & 0xFFFF

        def _unpack_dst(r, l0):
            for j in range(_CH // 16):
                p = idx_v[r, pl.ds(l0 + j * 16, 16)]
                dstg[pl.ds(j * 16, 16)] = p >> 16

        for j in range(nz):
            pltpu.make_async_copy(
                zbuf, agg_sh.at[pl.ds(base + j * _ZR, _ZR)], zsem).wait()

        @pl.when(s == 15)
        def _():
            for j in range(nz_rem):
                pltpu.make_async_copy(
                    zbuf, agg_sh.at[pl.ds(16 * rpt + j * _ZR, _ZR)],
                    zsem).wait()
        plsc.subcore_barrier()

        base_row = jnp.where(c == 0, s * _R0, 16 * _R0 + s * _R1)
        hrows = jnp.where(c == 0, h0, h1)
        ngh = jnp.where(c == 0, ng0, ng1)

        for t in range(_HV):
            half_row = base_row + t * hrows

            @pl.when(c == 0)
            def _(half_row=half_row):
                pltpu.sync_copy(idx_hbm.at[pl.ds(half_row, h0)],
                                idx_v.at[pl.ds(0, h0)])

            @pl.when(c == 1)
            def _(half_row=half_row):
                pltpu.sync_copy(idx_hbm.at[pl.ds(half_row, h1)],
                                idx_v.at[pl.ds(0, h1)])

            for b in range(_NB):
                _unpack_src(b // cpr, (b % cpr) * _CH, b)
                pltpu.async_copy(h_hbm.at[sstg[b]], rows[b], sems[b])

            def _group(g, carry):
                for b in range(_NB):
                    r = (_NB // cpr) * g + b // cpr
                    l0 = (b % cpr) * _CH
                    pltpu.make_async_copy(
                        h_hbm.at[sstg[b]], rows[b], sems[b]).wait()
                    _unpack_dst(r, l0)
                    pltpu.sync_copy(rows[b], agg_sh.at[dstg], add=True)

                    @pl.when(g < ngh - 1)
                    def _(r=r, l0=l0, b=b):
                        _unpack_src(r + _NB // cpr, l0, b)
                        pltpu.async_copy(
                            h_hbm.at[sstg[b]], rows[b], sems[b])
                return carry
            lax.fori_loop(0, ngh, _group, 0)

        plsc.subcore_barrier()

        @pl.when(s < 15)
        def _():
            pltpu.sync_copy(agg_sh.at[pl.ds(base, rpt)],
                            out_hbm.at[c].at[pl.ds(base, rpt)])

        @pl.when(s == 15)
        def _():
            pltpu.sync_copy(agg_sh.at[pl.ds(base, rpt + rem_out)],
                            out_hbm.at[c].at[pl.ds(base, rpt + rem_out)])

    return sc_scatter


def kernel(x, params, edge_index, batch):
    n, d = x.shape
    e = edge_index.shape[1]
    blk = 2000

    e_pad = 16 * (_R0 + _R1) * 128
    pad = e_pad - e
    src = edge_index[0].astype(jnp.int32)
    dst = edge_index[1].astype(jnp.int32)
    pad_iota = jnp.arange(pad, dtype=jnp.int32)
    src = jnp.concatenate([src, pad_iota % n])
    dst = jnp.concatenate([dst, n + (pad_iota % _PAD_ROWS)])
    packed = (src | (dst << 16)).reshape(16 * (_R0 + _R1), 128)
    batch3 = batch.astype(jnp.int32).reshape(n // blk, 1, blk)

    h = _premlp(x,
                params["pre1_w"], params["pre1_b"].reshape(1, -1),
                params["pre2_w"], params["pre2_b"].reshape(1, -1), blk)

    sc_scatter = _make_sc_scatter(n, d)
    pooled = []
    for lp in params["layers"]:
        agg = sc_scatter(h, packed)
        h, p = _layer_tc(h, agg, batch3, lp["eps"].reshape(1, 1),
                         lp["w1"], lp["b1"].reshape(1, -1),
                         lp["w2"], lp["b2"].reshape(1, -1), blk)
        pooled.append(p)
    return jnp.concatenate(pooled, axis=1)


# speedup vs baseline: 4.6381x; 1.0259x over previous
import functools

import jax
import jax.numpy as jnp
from jax import lax
from jax.experimental import pallas as pl
from jax.experimental.pallas import tpu as pltpu
from jax.experimental.pallas import tpu_sc as plsc

_G = 16
_NTILES = 32
_CH = 64
_NB = 4
_ZR = 16
_R0 = 80
_R1 = 80
_HV = 1
_PAD_ROWS = 64


def _premlp_body(x_ref, w1_ref, b1_ref, w2_ref, b2_ref, o_ref):
    h = jnp.dot(x_ref[...], w1_ref[...], preferred_element_type=jnp.float32)
    h = jnp.maximum(h + b1_ref[...], 0.0)
    o = jnp.dot(h, w2_ref[...], preferred_element_type=jnp.float32)
    o_ref[...] = jnp.maximum(o + b2_ref[...], 0.0)


def _premlp(x, w1, b1, w2, b2, blk):
    n, d = x.shape
    h = w1.shape[1]
    return pl.pallas_call(
        _premlp_body,
        grid=(n // blk,),
        in_specs=[
            pl.BlockSpec((blk, d), lambda i: (i, 0)),
            pl.BlockSpec((d, h), lambda i: (0, 0)),
            pl.BlockSpec((1, h), lambda i: (0, 0)),
            pl.BlockSpec((h, h), lambda i: (0, 0)),
            pl.BlockSpec((1, h), lambda i: (0, 0)),
        ],
        out_specs=pl.BlockSpec((blk, h), lambda i: (i, 0)),
        out_shape=jax.ShapeDtypeStruct((n, h), jnp.float32),
    )(x, w1, b1, w2, b2)


def _layer_body(h_ref, agg_ref, batch_ref, eps_ref, w1_ref, b1_ref, w2_ref,
                b2_ref, hout_ref, pool_ref):
    i = pl.program_id(0)
    eps = eps_ref[0, 0]
    a = (1.0 + eps) * h_ref[...] + agg_ref[0] + agg_ref[1]
    z = jnp.dot(a, w1_ref[...], preferred_element_type=jnp.float32)
    z = jnp.maximum(z + b1_ref[...], 0.0)
    hn = jnp.dot(z, w2_ref[...], preferred_element_type=jnp.float32) + b2_ref[...]
    hout_ref[...] = hn
    blk = h_ref.shape[0]
    seg = batch_ref[0]
    ids = lax.broadcasted_iota(jnp.int32, (_G, blk), 0)
    m = (seg == ids).astype(jnp.float32)
    p = jnp.dot(m, hn, preferred_element_type=jnp.float32)

    @pl.when(i == 0)
    def _():
        pool_ref[...] = p

    @pl.when(i != 0)
    def _():
        pool_ref[...] += p


def _layer_tc(h, agg, batch3, eps, w1, b1, w2, b2, blk):
    n, d = h.shape
    return pl.pallas_call(
        _layer_body,
        grid=(n // blk,),
        in_specs=[
            pl.BlockSpec((blk, d), lambda i: (i, 0)),
            pl.BlockSpec((2, blk, d), lambda i: (0, i, 0)),
            pl.BlockSpec((1, 1, blk), lambda i: (i, 0, 0)),
            pl.BlockSpec((1, 1), lambda i: (0, 0)),
            pl.BlockSpec((d, d), lambda i: (0, 0)),
            pl.BlockSpec((1, d), lambda i: (0, 0)),
            pl.BlockSpec((d, d), lambda i: (0, 0)),
            pl.BlockSpec((1, d), lambda i: (0, 0)),
        ],
        out_specs=[
            pl.BlockSpec((blk, d), lambda i: (i, 0)),
            pl.BlockSpec((_G, d), lambda i: (0, 0)),
        ],
        out_shape=[
            jax.ShapeDtypeStruct((n, d), jnp.float32),
            jax.ShapeDtypeStruct((_G, d), jnp.float32),
        ],
    )(h, agg, batch3, eps, w1, b1, w2, b2)


def _make_sc_scatter(n, d):
    h0 = _R0 // _HV
    h1 = _R1 // _HV
    ng0 = h0 * 2 // _NB
    ng1 = h1 * 2 // _NB
    cpr = 128 // _CH
    assert h0 * _HV == _R0 and ng0 * _NB == h0 * 2
    assert h1 * _HV == _R1 and ng1 * _NB == h1 * 2
    assert h0 % 8 == 0 and h1 % 8 == 0 and _NB % cpr == 0
    assert d % 16 == 0 and _CH % 16 == 0
    n_acc = n + _PAD_ROWS
    rpt = (n // 16) // 8 * 8
    rem_out = n - 16 * rpt
    rem_z = n_acc - 16 * rpt
    nz = rpt // _ZR
    assert rpt % _ZR == 0 and rem_z % _ZR == 0
    nz_rem = rem_z // _ZR
    mesh = plsc.VectorSubcoreMesh(core_axis_name="c", subcore_axis_name="s")

    @functools.partial(
        pl.kernel,
        mesh=mesh,
        out_type=jax.ShapeDtypeStruct((2, n, d), jnp.float32),
        scratch_types=[
            pltpu.VMEM((_R0 // _HV, 128), jnp.int32),
        ] + [pltpu.VMEM((_CH, d), jnp.float32) for _ in range(_NB)]
        + [pltpu.VMEM((_CH,), jnp.int32) for _ in range(_NB)]
        + [pltpu.VMEM((_CH,), jnp.int32),
           pltpu.VMEM((_ZR, d), jnp.float32),
           pltpu.VMEM_SHARED((n_acc, d), jnp.float32)]
        + [pltpu.SemaphoreType.DMA for _ in range(_NB)]
        + [pltpu.SemaphoreType.DMA],
    )
    def sc_scatter(h_hbm, idx_hbm, out_hbm, idx_v, *refs):
        rows = refs[:_NB]
        sstg = refs[_NB:2 * _NB]
        dstg, zbuf, agg_sh = refs[2 * _NB:2 * _NB + 3]
        sems = refs[2 * _NB + 3:3 * _NB + 3]
        zsem = refs[3 * _NB + 3]
        c = lax.axis_index("c")
        s = lax.axis_index("s")
        base = s * rpt

        def _zrow(r, carry):
            for j in range(d // 16):
                zbuf[r, pl.ds(j * 16, 16)] = jnp.zeros((16,), jnp.float32)
            return carry
        lax.fori_loop(0, _ZR, _zrow, 0)
        for j in range(nz):
            pltpu.async_copy(zbuf, agg_sh.at[pl.ds(base + j * _ZR, _ZR)],
                             zsem)

        @pl.when(s == 15)
        def _():
            for j in range(nz_rem):
                pltpu.async_copy(
                    zbuf, agg_sh.at[pl.ds(16 * rpt + j * _ZR, _ZR)], zsem)

        def _unpack_src(r, l0, b):
            for j in range(_CH // 16):
                p = idx_v[r, pl.ds(l0 + j * 16, 16)]
                sstg[b][pl.ds(j * 16, 16)] = p & 0xFFFF

        def _unpack_dst(r, l0):
            for j in range(_CH // 16):
                p = idx_v[r, pl.ds(l0 + j * 16, 16)]
                dstg[pl.ds(j * 16, 16)] = p >> 16

        for j in range(nz):
            pltpu.make_async_copy(
                zbuf, agg_sh.at[pl.ds(base + j * _ZR, _ZR)], zsem).wait()

        @pl.when(s == 15)
        def _():
            for j in range(nz_rem):
                pltpu.make_async_copy(
                    zbuf, agg_sh.at[pl.ds(16 * rpt + j * _ZR, _ZR)],
                    zsem).wait()
        plsc.subcore_barrier()

        base_row = jnp.where(c == 0, s * _R0, 16 * _R0 + s * _R1)
        hrows = jnp.where(c == 0, h0, h1)
        ngh = jnp.where(c == 0, ng0, ng1)

        for t in range(_HV):
            half_row = base_row + t * hrows

            @pl.when(c == 0)
            def _(half_row=half_row):
                pltpu.sync_copy(idx_hbm.at[pl.ds(half_row, h0)],
                                idx_v.at[pl.ds(0, h0)])

            @pl.when(c == 1)
            def _(half_row=half_row):
                pltpu.sync_copy(idx_hbm.at[pl.ds(half_row, h1)],
                                idx_v.at[pl.ds(0, h1)])

            for b in range(_NB):
                _unpack_src(b // cpr, (b % cpr) * _CH, b)
                pltpu.async_copy(h_hbm.at[sstg[b]], rows[b], sems[b])

            def _group(g, carry):
                for b in range(_NB):
                    r = (_NB // cpr) * g + b // cpr
                    l0 = (b % cpr) * _CH
                    pltpu.make_async_copy(
                        h_hbm.at[sstg[b]], rows[b], sems[b]).wait()
                    _unpack_dst(r, l0)
                    pltpu.sync_copy(rows[b], agg_sh.at[dstg], add=True)

                    @pl.when(g < ngh - 1)
                    def _(r=r, l0=l0, b=b):
                        _unpack_src(r + _NB // cpr, l0, b)
                        pltpu.async_copy(
                            h_hbm.at[sstg[b]], rows[b], sems[b])
                return carry
            lax.fori_loop(0, ngh, _group, 0)

        plsc.subcore_barrier()

        @pl.when(s < 15)
        def _():
            pltpu.sync_copy(agg_sh.at[pl.ds(base, rpt)],
                            out_hbm.at[c].at[pl.ds(base, rpt)])

        @pl.when(s == 15)
        def _():
            pltpu.sync_copy(agg_sh.at[pl.ds(base, rpt + rem_out)],
                            out_hbm.at[c].at[pl.ds(base, rpt + rem_out)])

    return sc_scatter


def kernel(x, params, edge_index, batch):
    n, d = x.shape
    e = edge_index.shape[1]
    blk = 2000

    e_pad = 16 * (_R0 + _R1) * 128
    pad = e_pad - e
    src = edge_index[0].astype(jnp.int32)
    dst = edge_index[1].astype(jnp.int32)
    pad_iota = jnp.arange(pad, dtype=jnp.int32)
    src = jnp.concatenate([src, pad_iota % n])
    dst = jnp.concatenate([dst, n + (pad_iota % _PAD_ROWS)])
    packed = (src | (dst << 16)).reshape(16 * (_R0 + _R1), 128)
    batch3 = batch.astype(jnp.int32).reshape(n // blk, 1, blk)

    h = _premlp(x,
                params["pre1_w"], params["pre1_b"].reshape(1, -1),
                params["pre2_w"], params["pre2_b"].reshape(1, -1), blk)

    sc_scatter = _make_sc_scatter(n, d)
    pooled = []
    for lp in params["layers"]:
        agg = sc_scatter(h, packed)
        h, p = _layer_tc(h, agg, batch3, lp["eps"].reshape(1, 1),
                         lp["w1"], lp["b1"].reshape(1, -1),
                         lp["w2"], lp["b2"].reshape(1, -1), blk)
        pooled.append(p)
    return jnp.concatenate(pooled, axis=1)


# speedup vs baseline: 4.6395x; 1.0003x over previous
import functools

import jax
import jax.numpy as jnp
from jax import lax
from jax.experimental import pallas as pl
from jax.experimental.pallas import tpu as pltpu
from jax.experimental.pallas import tpu_sc as plsc

_G = 16
_CH = 64
_NB = 4
_ZR = 16
_R0 = 80
_R1 = 80
_HV = 1
_PAD_ROWS = 64


def _premlp_body(x_ref, w1_ref, b1_ref, w2_ref, b2_ref, o_ref):
    h = jnp.dot(x_ref[...], w1_ref[...], preferred_element_type=jnp.float32)
    h = jnp.maximum(h + b1_ref[...], 0.0)
    o = jnp.dot(h, w2_ref[...], preferred_element_type=jnp.float32)
    o_ref[...] = jnp.maximum(o + b2_ref[...], 0.0)


def _premlp(x, w1, b1, w2, b2, blk):
    n, d = x.shape
    h = w1.shape[1]
    return pl.pallas_call(
        _premlp_body,
        grid=(n // blk,),
        in_specs=[
            pl.BlockSpec((blk, d), lambda i: (i, 0)),
            pl.BlockSpec((d, h), lambda i: (0, 0)),
            pl.BlockSpec((1, h), lambda i: (0, 0)),
            pl.BlockSpec((h, h), lambda i: (0, 0)),
            pl.BlockSpec((1, h), lambda i: (0, 0)),
        ],
        out_specs=pl.BlockSpec((blk, h), lambda i: (i, 0)),
        out_shape=jax.ShapeDtypeStruct((n, h), jnp.float32),
    )(x, w1, b1, w2, b2)


def _layer_body(h_ref, agg_ref, batch_ref, eps_ref, w1_ref, b1_ref, w2_ref,
                b2_ref, hout_ref, pool_ref):
    i = pl.program_id(0)
    eps = eps_ref[0, 0]
    a = (1.0 + eps) * h_ref[...] + agg_ref[0] + agg_ref[1]
    z = jnp.dot(a, w1_ref[...], preferred_element_type=jnp.float32)
    z = jnp.maximum(z + b1_ref[...], 0.0)
    hn = jnp.dot(z, w2_ref[...], preferred_element_type=jnp.float32) + b2_ref[...]
    hout_ref[...] = hn
    blk = h_ref.shape[0]
    seg = batch_ref[0]
    ids = lax.broadcasted_iota(jnp.int32, (_G, blk), 0)
    m = (seg == ids).astype(jnp.float32)
    p = jnp.dot(m, hn, preferred_element_type=jnp.float32)

    @pl.when(i == 0)
    def _():
        pool_ref[...] = p

    @pl.when(i != 0)
    def _():
        pool_ref[...] += p


def _layer_tc(h, agg, batch3, eps, w1, b1, w2, b2, blk):
    n, d = h.shape
    return pl.pallas_call(
        _layer_body,
        grid=(n // blk,),
        in_specs=[
            pl.BlockSpec((blk, d), lambda i: (i, 0)),
            pl.BlockSpec((2, blk, d), lambda i: (0, i, 0)),
            pl.BlockSpec((1, 1, blk), lambda i: (i, 0, 0)),
            pl.BlockSpec((1, 1), lambda i: (0, 0)),
            pl.BlockSpec((d, d), lambda i: (0, 0)),
            pl.BlockSpec((1, d), lambda i: (0, 0)),
            pl.BlockSpec((d, d), lambda i: (0, 0)),
            pl.BlockSpec((1, d), lambda i: (0, 0)),
        ],
        out_specs=[
            pl.BlockSpec((blk, d), lambda i: (i, 0)),
            pl.BlockSpec((_G, d), lambda i: (0, 0)),
        ],
        out_shape=[
            jax.ShapeDtypeStruct((n, d), jnp.float32),
            jax.ShapeDtypeStruct((_G, d), jnp.float32),
        ],
    )(h, agg, batch3, eps, w1, b1, w2, b2)


def _make_sc_scatter(n, d):
    h0 = _R0 // _HV
    h1 = _R1 // _HV
    ng0 = h0 * 2 // _NB
    ng1 = h1 * 2 // _NB
    cpr = 128 // _CH
    assert h0 * _HV == _R0 and ng0 * _NB == h0 * 2
    assert h1 * _HV == _R1 and ng1 * _NB == h1 * 2
    assert h0 % 8 == 0 and h1 % 8 == 0 and _NB % cpr == 0
    assert d % 16 == 0 and _CH % 16 == 0
    n_acc = n + _PAD_ROWS
    rpt = (n // 16) // 8 * 8
    rem_out = n - 16 * rpt
    rem_z = n_acc - 16 * rpt
    nz = rpt // _ZR
    assert rpt % _ZR == 0 and rem_z % _ZR == 0
    nz_rem = rem_z // _ZR
    mesh = plsc.VectorSubcoreMesh(core_axis_name="c", subcore_axis_name="s")

    @functools.partial(
        pl.kernel,
        mesh=mesh,
        out_type=jax.ShapeDtypeStruct((2, n, d), jnp.float32),
        scratch_types=[
            pltpu.VMEM((_R0 // _HV, 128), jnp.int32),
        ] + [pltpu.VMEM((_CH, d), jnp.float32) for _ in range(_NB)]
        + [pltpu.VMEM((_CH,), jnp.int32) for _ in range(_NB)]
        + [pltpu.VMEM((_CH,), jnp.int32),
           pltpu.VMEM((_ZR, d), jnp.float32),
           pltpu.VMEM_SHARED((n_acc, d), jnp.float32)]
        + [pltpu.SemaphoreType.DMA for _ in range(_NB)]
        + [pltpu.SemaphoreType.DMA],
    )
    def sc_scatter(h_hbm, idx_hbm, out_hbm, idx_v, *refs):
        rows = refs[:_NB]
        sstg = refs[_NB:2 * _NB]
        dstg, zbuf, agg_sh = refs[2 * _NB:2 * _NB + 3]
        sems = refs[2 * _NB + 3:3 * _NB + 3]
        zsem = refs[3 * _NB + 3]
        c = lax.axis_index("c")
        s = lax.axis_index("s")
        base = s * rpt

        def _zrow(r, carry):
            for j in range(d // 16):
                zbuf[r, pl.ds(j * 16, 16)] = jnp.zeros((16,), jnp.float32)
            return carry
        lax.fori_loop(0, _ZR, _zrow, 0)
        for j in range(nz):
            pltpu.async_copy(zbuf, agg_sh.at[pl.ds(base + j * _ZR, _ZR)],
                             zsem)

        @pl.when(s == 15)
        def _():
            for j in range(nz_rem):
                pltpu.async_copy(
                    zbuf, agg_sh.at[pl.ds(16 * rpt + j * _ZR, _ZR)], zsem)

        def _unpack_src(r, l0, b):
            for j in range(_CH // 16):
                p = idx_v[r, pl.ds(l0 + j * 16, 16)]
                sstg[b][pl.ds(j * 16, 16)] = p & 0xFFFF

        def _unpack_dst(r, l0):
            for j in range(_CH // 16):
                p = idx_v[r, pl.ds(l0 + j * 16, 16)]
                dstg[pl.ds(j * 16, 16)] = p >> 16

        for j in range(nz):
            pltpu.make_async_copy(
                zbuf, agg_sh.at[pl.ds(base + j * _ZR, _ZR)], zsem).wait()

        @pl.when(s == 15)
        def _():
            for j in range(nz_rem):
                pltpu.make_async_copy(
                    zbuf, agg_sh.at[pl.ds(16 * rpt + j * _ZR, _ZR)],
                    zsem).wait()
        plsc.subcore_barrier()

        base_row = jnp.where(c == 0, s * _R0, 16 * _R0 + s * _R1)
        hrows = jnp.where(c == 0, h0, h1)
        ngh = jnp.where(c == 0, ng0, ng1)

        for t in range(_HV):
            half_row = base_row + t * hrows

            @pl.when(c == 0)
            def _(half_row=half_row):
                pltpu.sync_copy(idx_hbm.at[pl.ds(half_row, h0)],
                                idx_v.at[pl.ds(0, h0)])

            @pl.when(c == 1)
            def _(half_row=half_row):
                pltpu.sync_copy(idx_hbm.at[pl.ds(half_row, h1)],
                                idx_v.at[pl.ds(0, h1)])

            for b in range(_NB):
                _unpack_src(b // cpr, (b % cpr) * _CH, b)
                pltpu.async_copy(h_hbm.at[sstg[b]], rows[b], sems[b])

            def _group(g, carry):
                for b in range(_NB):
                    r = (_NB // cpr) * g + b // cpr
                    l0 = (b % cpr) * _CH
                    pltpu.make_async_copy(
                        h_hbm.at[sstg[b]], rows[b], sems[b]).wait()
                    _unpack_dst(r, l0)
                    pltpu.sync_copy(rows[b], agg_sh.at[dstg], add=True)

                    @pl.when(g < ngh - 1)
                    def _(r=r, l0=l0, b=b):
                        _unpack_src(r + _NB // cpr, l0, b)
                        pltpu.async_copy(
                            h_hbm.at[sstg[b]], rows[b], sems[b])
                return carry
            lax.fori_loop(0, ngh, _group, 0)

        plsc.subcore_barrier()

        @pl.when(s < 15)
        def _():
            pltpu.sync_copy(agg_sh.at[pl.ds(base, rpt)],
                            out_hbm.at[c].at[pl.ds(base, rpt)])

        @pl.when(s == 15)
        def _():
            pltpu.sync_copy(agg_sh.at[pl.ds(base, rpt + rem_out)],
                            out_hbm.at[c].at[pl.ds(base, rpt + rem_out)])

    return sc_scatter


def kernel(x, params, edge_index, batch):
    n, d = x.shape
    e = edge_index.shape[1]
    blk = 2000

    e_pad = 16 * (_R0 + _R1) * 128
    pad = e_pad - e
    src = edge_index[0].astype(jnp.int32)
    dst = edge_index[1].astype(jnp.int32)
    pad_iota = jnp.arange(pad, dtype=jnp.int32)
    src = jnp.concatenate([src, pad_iota % n])
    dst = jnp.concatenate([dst, n + (pad_iota % _PAD_ROWS)])
    packed = (src | (dst << 16)).reshape(16 * (_R0 + _R1), 128)
    batch3 = batch.astype(jnp.int32).reshape(n // blk, 1, blk)

    h = _premlp(x,
                params["pre1_w"], params["pre1_b"].reshape(1, -1),
                params["pre2_w"], params["pre2_b"].reshape(1, -1), blk)

    sc_scatter = _make_sc_scatter(n, d)
    pooled = []
    for lp in params["layers"]:
        agg = sc_scatter(h, packed)
        h, p = _layer_tc(h, agg, batch3, lp["eps"].reshape(1, 1),
                         lp["w1"], lp["b1"].reshape(1, -1),
                         lp["w2"], lp["b2"].reshape(1, -1), blk)
        pooled.append(p)
    return jnp.concatenate(pooled, axis=1)
